# baseline (device time: 196450 ns/iter reference)
import jax
import jax.numpy as jnp
from jax import lax
from jax.experimental import pallas as pl
from jax.experimental.pallas import tpu as pltpu

N_DEV = 32
NZ = 4
NP = 8


def _gelu(z):
    return 0.5 * z * (1.0 + jnp.tanh(0.7978845608 * (z + 0.044715 * z * z * z)))


def kernel(A, B):
    m, k = A.shape
    k2, n = B.shape
    assert k == k2
    half = m // 2
    prow = half // NP
    zrow = prow // NZ

    def body(a_ref, b_ref, out_ref, pcp, pcm, zc, psp_s, psp_r, psm_s, psm_r,
             zs_s, zs_r):
        my = lax.axis_index("i")
        z = my // NP
        p = my % NP
        z_left = (my - NP) % N_DEV
        z_right = (my + NP) % N_DEV
        p_left = z * NP + (p - 1) % NP
        p_right = z * NP + (p + 1) % NP

        barrier_sem = pltpu.get_barrier_semaphore()
        for nbr in (z_left, z_right, p_left, p_right):
            pl.semaphore_signal(
                barrier_sem, inc=1,
                device_id=(nbr,), device_id_type=pl.DeviceIdType.MESH,
            )
        pl.semaphore_wait(barrier_sem, 4)

        out_ref[:, :] = jnp.dot(
            a_ref[:, :], b_ref[:, :], preferred_element_type=jnp.float32
        )

        def prow_p(q):
            return q * prow

        def prow_m(q):
            return half + q * prow

        for s in range(NP - 1):
            sq_p = (p - s) % NP
            rq_p = (p - s - 1) % NP
            sq_m = (p + s) % NP
            rq_m = (p + s + 1) % NP
            r_plus = pltpu.make_async_remote_copy(
                src_ref=out_ref.at[pl.ds(prow_p(sq_p), prow), :],
                dst_ref=pcp.at[s],
                send_sem=psp_s.at[s], recv_sem=psp_r.at[s],
                device_id=(p_right,), device_id_type=pl.DeviceIdType.MESH,
            )
            r_minus = pltpu.make_async_remote_copy(
                src_ref=out_ref.at[pl.ds(prow_m(sq_m), prow), :],
                dst_ref=pcm.at[s],
                send_sem=psm_s.at[s], recv_sem=psm_r.at[s],
                device_id=(p_left,), device_id_type=pl.DeviceIdType.MESH,
            )
            r_plus.start()
            r_plus.wait()
            out_ref[pl.ds(prow_p(rq_p), prow), :] += pcp[s]
            r_minus.start()
            r_minus.wait()
            out_ref[pl.ds(prow_m(rq_m), prow), :] += pcm[s]

        r_l = (p + 1) % NP
        r_r = (p - 1) % NP
        base_l = r_l * prow
        base_m = half + r_r * prow

        for s in range(NZ - 1):
            szc = (z - s) % NZ
            rzc = (z - s - 1) % NZ
            for piece, base in enumerate((base_l, base_m)):
                t = 2 * s + piece
                rd = pltpu.make_async_remote_copy(
                    src_ref=out_ref.at[pl.ds(base + szc * zrow, zrow), :],
                    dst_ref=zc.at[t],
                    send_sem=zs_s.at[t], recv_sem=zs_r.at[t],
                    device_id=(z_right,), device_id_type=pl.DeviceIdType.MESH,
                )
                rd.start()
                rd.wait()
                out_ref[pl.ds(base + rzc * zrow, zrow), :] += zc[t]

        zc_own = (z + 1) % NZ
        for base in (base_l, base_m):
            sl = pl.ds(base + zc_own * zrow, zrow)
            out_ref[sl, :] = _gelu(out_ref[sl, :])

        for s in range(NZ - 1):
            szc = (z + 1 - s) % NZ
            rzc = (z - s) % NZ
            for piece, base in enumerate((base_l, base_m)):
                t = 2 * (NZ - 1) + 2 * s + piece
                rd = pltpu.make_async_remote_copy(
                    src_ref=out_ref.at[pl.ds(base + szc * zrow, zrow), :],
                    dst_ref=zc.at[t],
                    send_sem=zs_s.at[t], recv_sem=zs_r.at[t],
                    device_id=(z_right,), device_id_type=pl.DeviceIdType.MESH,
                )
                rd.start()
                rd.wait()
                out_ref[pl.ds(base + rzc * zrow, zrow), :] = zc[t]

        for s in range(NP - 1):
            t = (NP - 1) + s
            sq_p = (p + 1 - s) % NP
            rq_p = (p - s) % NP
            sq_m = (p - 1 + s) % NP
            rq_m = (p + s) % NP
            r_plus = pltpu.make_async_remote_copy(
                src_ref=out_ref.at[pl.ds(prow_p(sq_p), prow), :],
                dst_ref=pcp.at[t],
                send_sem=psp_s.at[t], recv_sem=psp_r.at[t],
                device_id=(p_right,), device_id_type=pl.DeviceIdType.MESH,
            )
            r_minus = pltpu.make_async_remote_copy(
                src_ref=out_ref.at[pl.ds(prow_m(sq_m), prow), :],
                dst_ref=pcm.at[t],
                send_sem=psm_s.at[t], recv_sem=psm_r.at[t],
                device_id=(p_left,), device_id_type=pl.DeviceIdType.MESH,
            )
            r_plus.start()
            r_plus.wait()
            out_ref[pl.ds(prow_p(rq_p), prow), :] = pcp[t]
            r_minus.start()
            r_minus.wait()
            out_ref[pl.ds(prow_m(rq_m), prow), :] = pcm[t]

    n_p = 2 * (NP - 1)
    n_zsl = 4 * (NZ - 1)
    return pl.pallas_call(
        body,
        out_shape=jax.ShapeDtypeStruct((m, n), jnp.float32),
        in_specs=[
            pl.BlockSpec(memory_space=pltpu.VMEM),
            pl.BlockSpec(memory_space=pltpu.VMEM),
        ],
        out_specs=pl.BlockSpec(memory_space=pltpu.VMEM),
        scratch_shapes=[
            pltpu.VMEM((n_p, prow, n), jnp.float32),
            pltpu.VMEM((n_p, prow, n), jnp.float32),
            pltpu.VMEM((n_zsl, zrow, n), jnp.float32),
            pltpu.SemaphoreType.DMA((n_p,)),
            pltpu.SemaphoreType.DMA((n_p,)),
            pltpu.SemaphoreType.DMA((n_p,)),
            pltpu.SemaphoreType.DMA((n_p,)),
            pltpu.SemaphoreType.DMA((n_zsl,)),
            pltpu.SemaphoreType.DMA((n_zsl,)),
        ],
        compiler_params=pltpu.CompilerParams(collective_id=0),
    )(A, B)


# device time: 139090 ns/iter; 1.4124x vs baseline; 1.4124x over previous
import jax
import jax.numpy as jnp
from jax import lax
from jax.experimental import pallas as pl
from jax.experimental.pallas import tpu as pltpu

N_DEV = 32
NZ = 4
NP = 8


def _gelu(z):
    return 0.5 * z * (1.0 + jnp.tanh(0.7978845608 * (z + 0.044715 * z * z * z)))


def kernel(A, B):
    m, k = A.shape
    k2, n = B.shape
    assert k == k2
    half = m // 2
    prow = half // NP
    zrow = prow // NZ

    def body(a_ref, b_ref, out_ref, pcp, pcm, zc, psp_s, psp_r, psm_s, psm_r,
             zs_s, zs_r):
        my = lax.axis_index("i")
        z = my // NP
        p = my % NP
        z_left = (my - NP) % N_DEV
        z_right = (my + NP) % N_DEV
        p_left = z * NP + (p - 1) % NP
        p_right = z * NP + (p + 1) % NP

        barrier_sem = pltpu.get_barrier_semaphore()
        for nbr in (z_left, z_right, p_left, p_right):
            pl.semaphore_signal(
                barrier_sem, inc=1,
                device_id=(nbr,), device_id_type=pl.DeviceIdType.MESH,
            )
        pl.semaphore_wait(barrier_sem, 4)

        out_ref[:, :] = jnp.dot(
            a_ref[:, :], b_ref[:, :], preferred_element_type=jnp.float32
        )

        def prow_p(q):
            return q * prow

        def prow_m(q):
            return half + q * prow

        for s in range(NP - 1):
            sq_p = (p - s) % NP
            rq_p = (p - s - 1) % NP
            sq_m = (p + s) % NP
            rq_m = (p + s + 1) % NP
            r_plus = pltpu.make_async_remote_copy(
                src_ref=out_ref.at[pl.ds(prow_p(sq_p), prow), :],
                dst_ref=pcp.at[s],
                send_sem=psp_s.at[s], recv_sem=psp_r.at[s],
                device_id=(p_right,), device_id_type=pl.DeviceIdType.MESH,
            )
            r_minus = pltpu.make_async_remote_copy(
                src_ref=out_ref.at[pl.ds(prow_m(sq_m), prow), :],
                dst_ref=pcm.at[s],
                send_sem=psm_s.at[s], recv_sem=psm_r.at[s],
                device_id=(p_left,), device_id_type=pl.DeviceIdType.MESH,
            )
            r_plus.start()
            r_minus.start()
            r_plus.wait()
            out_ref[pl.ds(prow_p(rq_p), prow), :] += pcp[s]
            r_minus.wait()
            out_ref[pl.ds(prow_m(rq_m), prow), :] += pcm[s]

        r_l = (p + 1) % NP
        r_r = (p - 1) % NP
        base_l = r_l * prow
        base_m = half + r_r * prow

        for s in range(NZ - 1):
            szc = (z - s) % NZ
            rzc = (z - s - 1) % NZ
            for piece, base in enumerate((base_l, base_m)):
                t = 2 * s + piece
                rd = pltpu.make_async_remote_copy(
                    src_ref=out_ref.at[pl.ds(base + szc * zrow, zrow), :],
                    dst_ref=zc.at[t],
                    send_sem=zs_s.at[t], recv_sem=zs_r.at[t],
                    device_id=(z_right,), device_id_type=pl.DeviceIdType.MESH,
                )
                rd.start()
                rd.wait()
                out_ref[pl.ds(base + rzc * zrow, zrow), :] += zc[t]

        zc_own = (z + 1) % NZ
        for base in (base_l, base_m):
            sl = pl.ds(base + zc_own * zrow, zrow)
            out_ref[sl, :] = _gelu(out_ref[sl, :])

        for s in range(NZ - 1):
            szc = (z + 1 - s) % NZ
            rzc = (z - s) % NZ
            for piece, base in enumerate((base_l, base_m)):
                t = 2 * (NZ - 1) + 2 * s + piece
                rd = pltpu.make_async_remote_copy(
                    src_ref=out_ref.at[pl.ds(base + szc * zrow, zrow), :],
                    dst_ref=zc.at[t],
                    send_sem=zs_s.at[t], recv_sem=zs_r.at[t],
                    device_id=(z_right,), device_id_type=pl.DeviceIdType.MESH,
                )
                rd.start()
                rd.wait()
                out_ref[pl.ds(base + rzc * zrow, zrow), :] = zc[t]

        for s in range(NP - 1):
            t = (NP - 1) + s
            sq_p = (p + 1 - s) % NP
            rq_p = (p - s) % NP
            sq_m = (p - 1 + s) % NP
            rq_m = (p + s) % NP
            r_plus = pltpu.make_async_remote_copy(
                src_ref=out_ref.at[pl.ds(prow_p(sq_p), prow), :],
                dst_ref=pcp.at[t],
                send_sem=psp_s.at[t], recv_sem=psp_r.at[t],
                device_id=(p_right,), device_id_type=pl.DeviceIdType.MESH,
            )
            r_minus = pltpu.make_async_remote_copy(
                src_ref=out_ref.at[pl.ds(prow_m(sq_m), prow), :],
                dst_ref=pcm.at[t],
                send_sem=psm_s.at[t], recv_sem=psm_r.at[t],
                device_id=(p_left,), device_id_type=pl.DeviceIdType.MESH,
            )
            r_plus.start()
            r_minus.start()
            r_plus.wait()
            out_ref[pl.ds(prow_p(rq_p), prow), :] = pcp[t]
            r_minus.wait()
            out_ref[pl.ds(prow_m(rq_m), prow), :] = pcm[t]

    n_p = 2 * (NP - 1)
    n_zsl = 4 * (NZ - 1)
    return pl.pallas_call(
        body,
        out_shape=jax.ShapeDtypeStruct((m, n), jnp.float32),
        in_specs=[
            pl.BlockSpec(memory_space=pltpu.VMEM),
            pl.BlockSpec(memory_space=pltpu.VMEM),
        ],
        out_specs=pl.BlockSpec(memory_space=pltpu.VMEM),
        scratch_shapes=[
            pltpu.VMEM((n_p, prow, n), jnp.float32),
            pltpu.VMEM((n_p, prow, n), jnp.float32),
            pltpu.VMEM((n_zsl, zrow, n), jnp.float32),
            pltpu.SemaphoreType.DMA((n_p,)),
            pltpu.SemaphoreType.DMA((n_p,)),
            pltpu.SemaphoreType.DMA((n_p,)),
            pltpu.SemaphoreType.DMA((n_p,)),
            pltpu.SemaphoreType.DMA((n_zsl,)),
            pltpu.SemaphoreType.DMA((n_zsl,)),
        ],
        compiler_params=pltpu.CompilerParams(collective_id=0),
    )(A, B)


# device time: 96651 ns/iter; 2.0326x vs baseline; 1.4391x over previous
import jax
import jax.numpy as jnp
from jax import lax
from jax.experimental import pallas as pl
from jax.experimental.pallas import tpu as pltpu

N_DEV = 32
NZ = 4
NP = 8

_P_OF_R = (0, 1, 2, 5, 6, 7, 4, 3)
_R_OF_P = (0, 1, 2, 7, 6, 3, 4, 5)
_NEXT_P = (1, 2, 5, 0, 3, 6, 7, 4)
_PREV_P = (3, 0, 1, 4, 7, 2, 5, 6)


def _lut(table, idx):
    res = jnp.int32(table[0])
    for i in range(1, len(table)):
        res = jnp.where(idx == i, jnp.int32(table[i]), res)
    return res


def _gelu(z):
    return 0.5 * z * (1.0 + jnp.tanh(0.7978845608 * (z + 0.044715 * z * z * z)))


def kernel(A, B):
    m, k = A.shape
    k2, n = B.shape
    assert k == k2
    half = m // 2
    prow = half // NP
    zrow = prow // NZ
    zslab = 2 * zrow

    def body(a_ref, b_ref, out_ref, pcp, pcm, zw, zc, psp_s, psp_r, psm_s,
             psm_r, zs_s, zs_r):
        my = lax.axis_index("i")
        z = my // NP
        p = my % NP
        r = _lut(_R_OF_P, p)
        z_left = (my - NP) % N_DEV
        z_right = (my + NP) % N_DEV
        p_right = z * NP + _lut(_NEXT_P, p)
        p_left = z * NP + _lut(_PREV_P, p)

        barrier_sem = pltpu.get_barrier_semaphore()
        for nbr in (z_left, z_right, p_left, p_right):
            pl.semaphore_signal(
                barrier_sem, inc=1,
                device_id=(nbr,), device_id_type=pl.DeviceIdType.MESH,
            )
        pl.semaphore_wait(barrier_sem, 4)

        out_ref[:, :] = jnp.dot(
            a_ref[:, :], b_ref[:, :], preferred_element_type=jnp.float32
        )

        def row_p(c):
            return c * prow

        def row_m(c):
            return half + c * prow

        for s in range(NP - 1):
            sc_p = (r - s) % NP
            rc_p = (r - s - 1) % NP
            sc_m = (r + s) % NP
            rc_m = (r + s + 1) % NP
            r_plus = pltpu.make_async_remote_copy(
                src_ref=out_ref.at[pl.ds(row_p(sc_p), prow), :],
                dst_ref=pcp.at[s],
                send_sem=psp_s.at[s], recv_sem=psp_r.at[s],
                device_id=(p_right,), device_id_type=pl.DeviceIdType.MESH,
            )
            r_minus = pltpu.make_async_remote_copy(
                src_ref=out_ref.at[pl.ds(row_m(sc_m), prow), :],
                dst_ref=pcm.at[s],
                send_sem=psm_s.at[s], recv_sem=psm_r.at[s],
                device_id=(p_left,), device_id_type=pl.DeviceIdType.MESH,
            )
            r_plus.start()
            r_minus.start()
            r_plus.wait()
            out_ref[pl.ds(row_p(rc_p), prow), :] += pcp[s]
            r_minus.wait()
            out_ref[pl.ds(row_m(rc_m), prow), :] += pcm[s]

        c_l = (r + 1) % NP
        c_r = (r - 1) % NP
        base_l = c_l * prow
        base_m = half + c_r * prow

        for j in range(NZ):
            zw[j, :zrow, :] = out_ref[pl.ds(base_l + j * zrow, zrow), :]
            zw[j, zrow:, :] = out_ref[pl.ds(base_m + j * zrow, zrow), :]

        for s in range(NZ - 1):
            szc = (z - s) % NZ
            rzc = (z - s - 1) % NZ
            rd = pltpu.make_async_remote_copy(
                src_ref=zw.at[szc],
                dst_ref=zc.at[s],
                send_sem=zs_s.at[s], recv_sem=zs_r.at[s],
                device_id=(z_right,), device_id_type=pl.DeviceIdType.MESH,
            )
            rd.start()
            rd.wait()
            zw[rzc, :, :] += zc[s]

        zc_own = (z + 1) % NZ
        zw[zc_own, :, :] = _gelu(zw[zc_own, :, :])

        for s in range(NZ - 1):
            t = (NZ - 1) + s
            szc = (z + 1 - s) % NZ
            rzc = (z - s) % NZ
            rd = pltpu.make_async_remote_copy(
                src_ref=zw.at[szc],
                dst_ref=zc.at[t],
                send_sem=zs_s.at[t], recv_sem=zs_r.at[t],
                device_id=(z_right,), device_id_type=pl.DeviceIdType.MESH,
            )
            rd.start()
            rd.wait()
            zw[rzc, :, :] = zc[t]

        for j in range(NZ):
            out_ref[pl.ds(base_l + j * zrow, zrow), :] = zw[j, :zrow, :]
            out_ref[pl.ds(base_m + j * zrow, zrow), :] = zw[j, zrow:, :]

        for s in range(NP - 1):
            t = (NP - 1) + s
            sc_p = (r + 1 - s) % NP
            rc_p = (r - s) % NP
            sc_m = (r - 1 + s) % NP
            rc_m = (r + s) % NP
            r_plus = pltpu.make_async_remote_copy(
                src_ref=out_ref.at[pl.ds(row_p(sc_p), prow), :],
                dst_ref=pcp.at[t],
                send_sem=psp_s.at[t], recv_sem=psp_r.at[t],
                device_id=(p_right,), device_id_type=pl.DeviceIdType.MESH,
            )
            r_minus = pltpu.make_async_remote_copy(
                src_ref=out_ref.at[pl.ds(row_m(sc_m), prow), :],
                dst_ref=pcm.at[t],
                send_sem=psm_s.at[t], recv_sem=psm_r.at[t],
                device_id=(p_left,), device_id_type=pl.DeviceIdType.MESH,
            )
            r_plus.start()
            r_minus.start()
            r_plus.wait()
            out_ref[pl.ds(row_p(rc_p), prow), :] = pcp[t]
            r_minus.wait()
            out_ref[pl.ds(row_m(rc_m), prow), :] = pcm[t]

    n_p = 2 * (NP - 1)
    n_zs = 2 * (NZ - 1)
    return pl.pallas_call(
        body,
        out_shape=jax.ShapeDtypeStruct((m, n), jnp.float32),
        in_specs=[
            pl.BlockSpec(memory_space=pltpu.VMEM),
            pl.BlockSpec(memory_space=pltpu.VMEM),
        ],
        out_specs=pl.BlockSpec(memory_space=pltpu.VMEM),
        scratch_shapes=[
            pltpu.VMEM((n_p, prow, n), jnp.float32),
            pltpu.VMEM((n_p, prow, n), jnp.float32),
            pltpu.VMEM((NZ, zslab, n), jnp.float32),
            pltpu.VMEM((n_zs, zslab, n), jnp.float32),
            pltpu.SemaphoreType.DMA((n_p,)),
            pltpu.SemaphoreType.DMA((n_p,)),
            pltpu.SemaphoreType.DMA((n_p,)),
            pltpu.SemaphoreType.DMA((n_p,)),
            pltpu.SemaphoreType.DMA((n_zs,)),
            pltpu.SemaphoreType.DMA((n_zs,)),
        ],
        compiler_params=pltpu.CompilerParams(collective_id=0),
    )(A, B)


# device time: 96310 ns/iter; 2.0398x vs baseline; 1.0035x over previous
import jax
import jax.numpy as jnp
from jax import lax
from jax.experimental import pallas as pl
from jax.experimental.pallas import tpu as pltpu

N_DEV = 32
NZ = 4
NP = 8

_P_OF_R = (0, 1, 2, 5, 6, 7, 4, 3)
_R_OF_P = (0, 1, 2, 7, 6, 3, 4, 5)
_NEXT_P = (1, 2, 5, 0, 3, 6, 7, 4)
_PREV_P = (3, 0, 1, 4, 7, 2, 5, 6)


def _lut(table, idx):
    res = jnp.int32(table[0])
    for i in range(1, len(table)):
        res = jnp.where(idx == i, jnp.int32(table[i]), res)
    return res


def _gelu(z):
    return 0.5 * z * (1.0 + jnp.tanh(0.7978845608 * (z + 0.044715 * z * z * z)))


def kernel(A, B):
    m, k = A.shape
    k2, n = B.shape
    assert k == k2
    half = m // 2
    prow = half // NP
    zrow = prow // NZ
    zslab = 2 * zrow

    def body(a_ref, b_ref, out_ref, pcp, pcm, zw, zc, psp_s, psp_r, psm_s,
             psm_r, zs_s, zs_r):
        my = lax.axis_index("i")
        z = my // NP
        p = my % NP
        r = _lut(_R_OF_P, p)
        z_left = (my - NP) % N_DEV
        z_right = (my + NP) % N_DEV
        p_right = z * NP + _lut(_NEXT_P, p)
        p_left = z * NP + _lut(_PREV_P, p)

        barrier_sem = pltpu.get_barrier_semaphore()
        for nbr in (z_left, z_right, p_left, p_right):
            pl.semaphore_signal(
                barrier_sem, inc=1,
                device_id=(nbr,), device_id_type=pl.DeviceIdType.MESH,
            )
        pl.semaphore_wait(barrier_sem, 4)

        out_ref[:, :] = jnp.dot(
            a_ref[:, :], b_ref[:, :], preferred_element_type=jnp.float32
        )

        def row_p(c):
            return c * prow

        def row_m(c):
            return half + c * prow

        def mk_plus(s, chunk):
            return pltpu.make_async_remote_copy(
                src_ref=out_ref.at[pl.ds(row_p(chunk), prow), :],
                dst_ref=pcp.at[s],
                send_sem=psp_s.at[s], recv_sem=psp_r.at[s],
                device_id=(p_right,), device_id_type=pl.DeviceIdType.MESH,
            )

        def mk_minus(s, chunk):
            return pltpu.make_async_remote_copy(
                src_ref=out_ref.at[pl.ds(row_m(chunk), prow), :],
                dst_ref=pcm.at[s],
                send_sem=psm_s.at[s], recv_sem=psm_r.at[s],
                device_id=(p_left,), device_id_type=pl.DeviceIdType.MESH,
            )

        rs_sends = []
        rp = mk_plus(0, r % NP)
        rm = mk_minus(0, r % NP)
        rp.start()
        rm.start()
        for s in range(NP - 1):
            rc_p = (r - s - 1) % NP
            rc_m = (r + s + 1) % NP
            rp.wait_recv()
            out_ref[pl.ds(row_p(rc_p), prow), :] += pcp[s]
            rs_sends.append(rp)
            if s + 1 < NP - 1:
                rp = mk_plus(s + 1, rc_p)
                rp.start()
            rm.wait_recv()
            out_ref[pl.ds(row_m(rc_m), prow), :] += pcm[s]
            rs_sends.append(rm)
            if s + 1 < NP - 1:
                rm = mk_minus(s + 1, rc_m)
                rm.start()

        c_l = (r + 1) % NP
        c_r = (r - 1) % NP
        base_l = c_l * prow
        base_m = half + c_r * prow

        for j in range(NZ):
            zw[j, :zrow, :] = out_ref[pl.ds(base_l + j * zrow, zrow), :]
            zw[j, zrow:, :] = out_ref[pl.ds(base_m + j * zrow, zrow), :]

        z_sends = []
        for s in range(NZ - 1):
            szc = (z - s) % NZ
            rzc = (z - s - 1) % NZ
            rd = pltpu.make_async_remote_copy(
                src_ref=zw.at[szc],
                dst_ref=zc.at[s],
                send_sem=zs_s.at[s], recv_sem=zs_r.at[s],
                device_id=(z_right,), device_id_type=pl.DeviceIdType.MESH,
            )
            rd.start()
            rd.wait_recv()
            zw[rzc, :, :] += zc[s]
            z_sends.append(rd)

        zc_own = (z + 1) % NZ
        zw[zc_own, :, :] = _gelu(zw[zc_own, :, :])

        for rd in z_sends:
            rd.wait_send()

        zag_sends = []
        for s in range(NZ - 1):
            t = (NZ - 1) + s
            szc = (z + 1 - s) % NZ
            rzc = (z - s) % NZ
            rd = pltpu.make_async_remote_copy(
                src_ref=zw.at[szc],
                dst_ref=zc.at[t],
                send_sem=zs_s.at[t], recv_sem=zs_r.at[t],
                device_id=(z_right,), device_id_type=pl.DeviceIdType.MESH,
            )
            rd.start()
            rd.wait_recv()
            zw[rzc, :, :] = zc[t]
            zag_sends.append(rd)

        for j in range(NZ):
            out_ref[pl.ds(base_l + j * zrow, zrow), :] = zw[j, :zrow, :]
            out_ref[pl.ds(base_m + j * zrow, zrow), :] = zw[j, zrow:, :]

        for rd in rs_sends:
            rd.wait_send()

        ag_sends = []
        rp = mk_plus(NP - 1, c_l)
        rm = mk_minus(NP - 1, c_r)
        rp.start()
        rm.start()
        for s in range(NP - 1):
            t = (NP - 1) + s
            rc_p = (r - s) % NP
            rc_m = (r + s) % NP
            rp.wait_recv()
            out_ref[pl.ds(row_p(rc_p), prow), :] = pcp[t]
            ag_sends.append(rp)
            if s + 1 < NP - 1:
                rp = mk_plus(t + 1, rc_p)
                rp.start()
            rm.wait_recv()
            out_ref[pl.ds(row_m(rc_m), prow), :] = pcm[t]
            ag_sends.append(rm)
            if s + 1 < NP - 1:
                rm = mk_minus(t + 1, rc_m)
                rm.start()

        for rd in zag_sends + ag_sends:
            rd.wait_send()

    n_p = 2 * (NP - 1)
    n_zs = 2 * (NZ - 1)
    return pl.pallas_call(
        body,
        out_shape=jax.ShapeDtypeStruct((m, n), jnp.float32),
        in_specs=[
            pl.BlockSpec(memory_space=pltpu.VMEM),
            pl.BlockSpec(memory_space=pltpu.VMEM),
        ],
        out_specs=pl.BlockSpec(memory_space=pltpu.VMEM),
        scratch_shapes=[
            pltpu.VMEM((n_p, prow, n), jnp.float32),
            pltpu.VMEM((n_p, prow, n), jnp.float32),
            pltpu.VMEM((NZ, zslab, n), jnp.float32),
            pltpu.VMEM((n_zs, zslab, n), jnp.float32),
            pltpu.SemaphoreType.DMA((n_p,)),
            pltpu.SemaphoreType.DMA((n_p,)),
            pltpu.SemaphoreType.DMA((n_p,)),
            pltpu.SemaphoreType.DMA((n_p,)),
            pltpu.SemaphoreType.DMA((n_zs,)),
            pltpu.SemaphoreType.DMA((n_zs,)),
        ],
        compiler_params=pltpu.CompilerParams(collective_id=0),
    )(A, B)


# device time: 75884 ns/iter; 2.5888x vs baseline; 1.2692x over previous
import jax
import jax.numpy as jnp
from jax import lax
from jax.experimental import pallas as pl
from jax.experimental.pallas import tpu as pltpu

N_DEV = 32
NZ = 4
NP = 8

_R_OF_P = (0, 1, 2, 7, 6, 3, 4, 5)
_NEXT_P = (1, 2, 5, 0, 3, 6, 7, 4)
_PREV_P = (3, 0, 1, 4, 7, 2, 5, 6)


def _lut(table, idx):
    res = jnp.int32(table[0])
    for i in range(1, len(table)):
        res = jnp.where(idx == i, jnp.int32(table[i]), res)
    return res


def _gelu(z):
    return 0.5 * z * (1.0 + jnp.tanh(0.7978845608 * (z + 0.044715 * z * z * z)))


def kernel(A, B):
    m, k = A.shape
    k2, n = B.shape
    assert k == k2
    half = m // 2
    prow = half // NP
    hrow = prow // 2
    zrow = prow // NZ
    zslab = 2 * zrow

    def body(a_ref, b_ref, out_ref, pcp, pcm, zw, zc, psp_s, psp_r, psm_s,
             psm_r, zs_s, zs_r):
        my = lax.axis_index("i")
        z = my // NP
        p = my % NP
        r = _lut(_R_OF_P, p)
        z_left = (my - NP) % N_DEV
        z_right = (my + NP) % N_DEV
        p_right = z * NP + _lut(_NEXT_P, p)
        p_left = z * NP + _lut(_PREV_P, p)

        barrier_sem = pltpu.get_barrier_semaphore()
        for nbr in (z_left, z_right, p_left, p_right):
            pl.semaphore_signal(
                barrier_sem, inc=1,
                device_id=(nbr,), device_id_type=pl.DeviceIdType.MESH,
            )
        pl.semaphore_wait(barrier_sem, 4)

        out_ref[:, :] = jnp.dot(
            a_ref[:, :], b_ref[:, :], preferred_element_type=jnp.float32
        )

        def row_p(c):
            return c * prow

        def row_m(c):
            return half + c * prow

        def mk_plus(s, chunk, h):
            return pltpu.make_async_remote_copy(
                src_ref=out_ref.at[pl.ds(row_p(chunk) + h * hrow, hrow), :],
                dst_ref=pcp.at[s, pl.ds(h * hrow, hrow)],
                send_sem=psp_s.at[s, h], recv_sem=psp_r.at[s, h],
                device_id=(p_right,), device_id_type=pl.DeviceIdType.MESH,
            )

        def mk_minus(s, chunk, h):
            return pltpu.make_async_remote_copy(
                src_ref=out_ref.at[pl.ds(row_m(chunk) + h * hrow, hrow), :],
                dst_ref=pcm.at[s, pl.ds(h * hrow, hrow)],
                send_sem=psm_s.at[s, h], recv_sem=psm_r.at[s, h],
                device_id=(p_left,), device_id_type=pl.DeviceIdType.MESH,
            )

        rs_sends = []
        st = [mk_plus(0, r % NP, 0), mk_minus(0, r % NP, 0),
              mk_plus(0, r % NP, 1), mk_minus(0, r % NP, 1)]
        for rd in st:
            rd.start()
        for s in range(NP - 1):
            rc_p = (r - s - 1) % NP
            rc_m = (r + s + 1) % NP
            nxt = []
            for idx, rd in enumerate(st):
                plus = (idx % 2 == 0)
                h = idx // 2
                rd.wait_recv()
                if plus:
                    out_ref[pl.ds(row_p(rc_p) + h * hrow, hrow), :] += (
                        pcp[s, h * hrow:(h + 1) * hrow]
                    )
                else:
                    out_ref[pl.ds(row_m(rc_m) + h * hrow, hrow), :] += (
                        pcm[s, h * hrow:(h + 1) * hrow]
                    )
                rs_sends.append(rd)
                if s + 1 < NP - 1:
                    nrd = (mk_plus(s + 1, rc_p, h) if plus
                           else mk_minus(s + 1, rc_m, h))
                    nrd.start()
                    nxt.append(nrd)
            st = nxt

        c_l = (r + 1) % NP
        c_r = (r - 1) % NP
        base_l = c_l * prow
        base_m = half + c_r * prow

        for j in range(NZ):
            zw[j, :zrow, :] = out_ref[pl.ds(base_l + j * zrow, zrow), :]
            zw[j, zrow:, :] = out_ref[pl.ds(base_m + j * zrow, zrow), :]

        def mk_z(s, chunk, h):
            return pltpu.make_async_remote_copy(
                src_ref=zw.at[chunk, pl.ds(h * zrow, zrow)],
                dst_ref=zc.at[s, pl.ds(h * zrow, zrow)],
                send_sem=zs_s.at[s, h], recv_sem=zs_r.at[s, h],
                device_id=(z_right,), device_id_type=pl.DeviceIdType.MESH,
            )

        z_sends = []
        zst = [mk_z(0, z % NZ, 0), mk_z(0, z % NZ, 1)]
        for rd in zst:
            rd.start()
        for s in range(NZ - 1):
            rzc = (z - s - 1) % NZ
            nxt = []
            for h, rd in enumerate(zst):
                rd.wait_recv()
                zw[rzc, h * zrow:(h + 1) * zrow, :] += (
                    zc[s, h * zrow:(h + 1) * zrow]
                )
                z_sends.append(rd)
                if s + 1 < NZ - 1:
                    nrd = mk_z(s + 1, rzc, h)
                    nrd.start()
                    nxt.append(nrd)
            zst = nxt

        zc_own = (z + 1) % NZ
        zw[zc_own, :, :] = _gelu(zw[zc_own, :, :])

        for rd in z_sends:
            rd.wait_send()

        zag_sends = []
        zst = [mk_z(NZ - 1, zc_own, 0), mk_z(NZ - 1, zc_own, 1)]
        for rd in zst:
            rd.start()
        for s in range(NZ - 1):
            t = (NZ - 1) + s
            rzc = (z - s) % NZ
            nxt = []
            for h, rd in enumerate(zst):
                rd.wait_recv()
                zw[rzc, h * zrow:(h + 1) * zrow, :] = (
                    zc[t, h * zrow:(h + 1) * zrow]
                )
                zag_sends.append(rd)
                if s + 1 < NZ - 1:
                    nrd = mk_z(t + 1, rzc, h)
                    nrd.start()
                    nxt.append(nrd)
            zst = nxt

        for j in range(NZ):
            out_ref[pl.ds(base_l + j * zrow, zrow), :] = zw[j, :zrow, :]
            out_ref[pl.ds(base_m + j * zrow, zrow), :] = zw[j, zrow:, :]

        for rd in rs_sends:
            rd.wait_send()

        ag_sends = []
        st = [mk_plus(NP - 1, c_l, 0), mk_minus(NP - 1, c_r, 0),
              mk_plus(NP - 1, c_l, 1), mk_minus(NP - 1, c_r, 1)]
        for rd in st:
            rd.start()
        for s in range(NP - 1):
            t = (NP - 1) + s
            rc_p = (r - s) % NP
            rc_m = (r + s) % NP
            nxt = []
            for idx, rd in enumerate(st):
                plus = (idx % 2 == 0)
                h = idx // 2
                rd.wait_recv()
                if plus:
                    out_ref[pl.ds(row_p(rc_p) + h * hrow, hrow), :] = (
                        pcp[t, h * hrow:(h + 1) * hrow]
                    )
                else:
                    out_ref[pl.ds(row_m(rc_m) + h * hrow, hrow), :] = (
                        pcm[t, h * hrow:(h + 1) * hrow]
                    )
                ag_sends.append(rd)
                if s + 1 < NP - 1:
                    nrd = (mk_plus(t + 1, rc_p, h) if plus
                           else mk_minus(t + 1, rc_m, h))
                    nrd.start()
                    nxt.append(nrd)
            st = nxt

        for rd in zag_sends + ag_sends:
            rd.wait_send()

    n_p = 2 * (NP - 1)
    n_zs = 2 * (NZ - 1)
    return pl.pallas_call(
        body,
        out_shape=jax.ShapeDtypeStruct((m, n), jnp.float32),
        in_specs=[
            pl.BlockSpec(memory_space=pltpu.VMEM),
            pl.BlockSpec(memory_space=pltpu.VMEM),
        ],
        out_specs=pl.BlockSpec(memory_space=pltpu.VMEM),
        scratch_shapes=[
            pltpu.VMEM((n_p, prow, n), jnp.float32),
            pltpu.VMEM((n_p, prow, n), jnp.float32),
            pltpu.VMEM((NZ, zslab, n), jnp.float32),
            pltpu.VMEM((n_zs, zslab, n), jnp.float32),
            pltpu.SemaphoreType.DMA((n_p, 2)),
            pltpu.SemaphoreType.DMA((n_p, 2)),
            pltpu.SemaphoreType.DMA((n_p, 2)),
            pltpu.SemaphoreType.DMA((n_p, 2)),
            pltpu.SemaphoreType.DMA((n_zs, 2)),
            pltpu.SemaphoreType.DMA((n_zs, 2)),
        ],
        compiler_params=pltpu.CompilerParams(collective_id=0),
    )(A, B)


# device time: 72892 ns/iter; 2.6951x vs baseline; 1.0410x over previous
import jax
import jax.numpy as jnp
from jax import lax
from jax.experimental import pallas as pl
from jax.experimental.pallas import tpu as pltpu

N_DEV = 32
NZ = 4
NP = 8

_R_OF_P = (0, 1, 2, 7, 6, 3, 4, 5)
_NEXT_P = (1, 2, 5, 0, 3, 6, 7, 4)
_PREV_P = (3, 0, 1, 4, 7, 2, 5, 6)


def _lut(table, idx):
    res = jnp.int32(table[0])
    for i in range(1, len(table)):
        res = jnp.where(idx == i, jnp.int32(table[i]), res)
    return res


def _gelu(z):
    return 0.5 * z * (1.0 + jnp.tanh(0.7978845608 * (z + 0.044715 * z * z * z)))


def kernel(A, B):
    m, k = A.shape
    k2, n = B.shape
    assert k == k2
    half = m // 2
    prow = half // NP
    NS = 4
    hrow = prow // NS
    zrow = prow // NZ
    zslab = 2 * zrow

    def body(a_ref, b_ref, out_ref, pcp, pcm, zw, zc, psp_s, psp_r, psm_s,
             psm_r, zs_s, zs_r):
        my = lax.axis_index("i")
        z = my // NP
        p = my % NP
        r = _lut(_R_OF_P, p)
        z_left = (my - NP) % N_DEV
        z_right = (my + NP) % N_DEV
        p_right = z * NP + _lut(_NEXT_P, p)
        p_left = z * NP + _lut(_PREV_P, p)

        barrier_sem = pltpu.get_barrier_semaphore()
        for nbr in (z_left, z_right, p_left, p_right):
            pl.semaphore_signal(
                barrier_sem, inc=1,
                device_id=(nbr,), device_id_type=pl.DeviceIdType.MESH,
            )
        pl.semaphore_wait(barrier_sem, 4)

        out_ref[:, :] = jnp.dot(
            a_ref[:, :], b_ref[:, :], preferred_element_type=jnp.float32
        )

        def row_p(c):
            return c * prow

        def row_m(c):
            return half + c * prow

        def mk_plus(s, chunk, h):
            return pltpu.make_async_remote_copy(
                src_ref=out_ref.at[pl.ds(row_p(chunk) + h * hrow, hrow), :],
                dst_ref=pcp.at[s, pl.ds(h * hrow, hrow)],
                send_sem=psp_s.at[s, h], recv_sem=psp_r.at[s, h],
                device_id=(p_right,), device_id_type=pl.DeviceIdType.MESH,
            )

        def mk_minus(s, chunk, h):
            return pltpu.make_async_remote_copy(
                src_ref=out_ref.at[pl.ds(row_m(chunk) + h * hrow, hrow), :],
                dst_ref=pcm.at[s, pl.ds(h * hrow, hrow)],
                send_sem=psm_s.at[s, h], recv_sem=psm_r.at[s, h],
                device_id=(p_left,), device_id_type=pl.DeviceIdType.MESH,
            )

        rs_sends = []
        st = []
        for h in range(NS):
            st += [mk_plus(0, r % NP, h), mk_minus(0, r % NP, h)]
        for rd in st:
            rd.start()
        for s in range(NP - 1):
            rc_p = (r - s - 1) % NP
            rc_m = (r + s + 1) % NP
            nxt = []
            for idx, rd in enumerate(st):
                plus = (idx % 2 == 0)
                h = idx // 2
                rd.wait_recv()
                if plus:
                    out_ref[pl.ds(row_p(rc_p) + h * hrow, hrow), :] += (
                        pcp[s, h * hrow:(h + 1) * hrow]
                    )
                else:
                    out_ref[pl.ds(row_m(rc_m) + h * hrow, hrow), :] += (
                        pcm[s, h * hrow:(h + 1) * hrow]
                    )
                rs_sends.append(rd)
                if s + 1 < NP - 1:
                    nrd = (mk_plus(s + 1, rc_p, h) if plus
                           else mk_minus(s + 1, rc_m, h))
                    nrd.start()
                    nxt.append(nrd)
            st = nxt

        c_l = (r + 1) % NP
        c_r = (r - 1) % NP
        base_l = c_l * prow
        base_m = half + c_r * prow

        for j in range(NZ):
            zw[j, :zrow, :] = out_ref[pl.ds(base_l + j * zrow, zrow), :]
            zw[j, zrow:, :] = out_ref[pl.ds(base_m + j * zrow, zrow), :]

        def mk_z(s, chunk, h):
            return pltpu.make_async_remote_copy(
                src_ref=zw.at[chunk, pl.ds(h * zrow, zrow)],
                dst_ref=zc.at[s, pl.ds(h * zrow, zrow)],
                send_sem=zs_s.at[s, h], recv_sem=zs_r.at[s, h],
                device_id=(z_right,), device_id_type=pl.DeviceIdType.MESH,
            )

        z_sends = []
        zst = [mk_z(0, z % NZ, 0), mk_z(0, z % NZ, 1)]
        for rd in zst:
            rd.start()
        for s in range(NZ - 1):
            rzc = (z - s - 1) % NZ
            nxt = []
            for h, rd in enumerate(zst):
                rd.wait_recv()
                zw[rzc, h * zrow:(h + 1) * zrow, :] += (
                    zc[s, h * zrow:(h + 1) * zrow]
                )
                z_sends.append(rd)
                if s + 1 < NZ - 1:
                    nrd = mk_z(s + 1, rzc, h)
                    nrd.start()
                    nxt.append(nrd)
            zst = nxt

        zc_own = (z + 1) % NZ
        zw[zc_own, :, :] = _gelu(zw[zc_own, :, :])

        for rd in z_sends:
            rd.wait_send()

        zag_sends = []
        zst = [mk_z(NZ - 1, zc_own, 0), mk_z(NZ - 1, zc_own, 1)]
        for rd in zst:
            rd.start()
        for s in range(NZ - 1):
            t = (NZ - 1) + s
            rzc = (z - s) % NZ
            nxt = []
            for h, rd in enumerate(zst):
                rd.wait_recv()
                zw[rzc, h * zrow:(h + 1) * zrow, :] = (
                    zc[t, h * zrow:(h + 1) * zrow]
                )
                zag_sends.append(rd)
                if s + 1 < NZ - 1:
                    nrd = mk_z(t + 1, rzc, h)
                    nrd.start()
                    nxt.append(nrd)
            zst = nxt

        for j in range(NZ):
            out_ref[pl.ds(base_l + j * zrow, zrow), :] = zw[j, :zrow, :]
            out_ref[pl.ds(base_m + j * zrow, zrow), :] = zw[j, zrow:, :]

        for rd in rs_sends:
            rd.wait_send()

        ag_sends = []
        st = []
        for h in range(NS):
            st += [mk_plus(NP - 1, c_l, h), mk_minus(NP - 1, c_r, h)]
        for rd in st:
            rd.start()
        for s in range(NP - 1):
            t = (NP - 1) + s
            rc_p = (r - s) % NP
            rc_m = (r + s) % NP
            nxt = []
            for idx, rd in enumerate(st):
                plus = (idx % 2 == 0)
                h = idx // 2
                rd.wait_recv()
                if plus:
                    out_ref[pl.ds(row_p(rc_p) + h * hrow, hrow), :] = (
                        pcp[t, h * hrow:(h + 1) * hrow]
                    )
                else:
                    out_ref[pl.ds(row_m(rc_m) + h * hrow, hrow), :] = (
                        pcm[t, h * hrow:(h + 1) * hrow]
                    )
                ag_sends.append(rd)
                if s + 1 < NP - 1:
                    nrd = (mk_plus(t + 1, rc_p, h) if plus
                           else mk_minus(t + 1, rc_m, h))
                    nrd.start()
                    nxt.append(nrd)
            st = nxt

        for rd in zag_sends + ag_sends:
            rd.wait_send()

    n_p = 2 * (NP - 1)
    n_zs = 2 * (NZ - 1)
    return pl.pallas_call(
        body,
        out_shape=jax.ShapeDtypeStruct((m, n), jnp.float32),
        in_specs=[
            pl.BlockSpec(memory_space=pltpu.VMEM),
            pl.BlockSpec(memory_space=pltpu.VMEM),
        ],
        out_specs=pl.BlockSpec(memory_space=pltpu.VMEM),
        scratch_shapes=[
            pltpu.VMEM((n_p, prow, n), jnp.float32),
            pltpu.VMEM((n_p, prow, n), jnp.float32),
            pltpu.VMEM((NZ, zslab, n), jnp.float32),
            pltpu.VMEM((n_zs, zslab, n), jnp.float32),
            pltpu.SemaphoreType.DMA((n_p, NS)),
            pltpu.SemaphoreType.DMA((n_p, NS)),
            pltpu.SemaphoreType.DMA((n_p, NS)),
            pltpu.SemaphoreType.DMA((n_p, NS)),
            pltpu.SemaphoreType.DMA((n_zs, 2)),
            pltpu.SemaphoreType.DMA((n_zs, 2)),
        ],
        compiler_params=pltpu.CompilerParams(collective_id=0),
    )(A, B)
